# baseline (speedup 1.0000x reference)
import jax
import jax.numpy as jnp
from jax.experimental import pallas as pl
from jax.experimental.pallas import tpu as pltpu

B, N, DIN, NH, DH = 256, 128, 128, 8, 16
HID = NH * DH
DOUT = 512

_NEG = -9e15


def _leaky(x):
    return jnp.where(x > 0, x, 0.2 * x)


def _elu(x):
    return jnp.where(x > 0, x, jnp.exp(jnp.minimum(x, 0.0)) - 1.0)


def _masked_softmax(e, adj):
    e = jnp.where(adj > 0, e, _NEG)
    m = jnp.max(e, axis=-1, keepdims=True)
    p = jnp.exp(e - m)
    return p / jnp.sum(p, axis=-1, keepdims=True)


def _col_row(h, s_col, d_col):
    es = jnp.dot(h, s_col, preferred_element_type=jnp.float32)
    ed = jax.lax.dot_general(
        d_col, h, (((0,), (1,)), ((), ())),
        preferred_element_type=jnp.float32)
    return es, ed


def _gat_kernel(x_ref, adj_ref, W1c_ref, A1s_ref, A1d_ref,
                W2_ref, a2s_ref, a2d_ref, W3_ref, a3s_ref, a3d_ref,
                fcw_ref, fcb_ref, o_ref):
    x = x_ref[0]
    adj = adj_ref[0]

    h1 = jnp.dot(x, W1c_ref[...], preferred_element_type=jnp.float32)
    es = jnp.dot(h1, A1s_ref[...], preferred_element_type=jnp.float32)
    ed_t = jax.lax.dot_general(
        A1d_ref[...], h1, (((0,), (1,)), ((), ())),
        preferred_element_type=jnp.float32)
    outs = []
    for i in range(NH):
        e = _leaky(es[:, i:i + 1] + ed_t[i:i + 1, :])
        att = _masked_softmax(e, adj)
        outs.append(jnp.dot(att, h1[:, i * DH:(i + 1) * DH],
                            preferred_element_type=jnp.float32))
    mole = _elu(jnp.concatenate(outs, axis=1))

    h2 = jnp.dot(mole, W2_ref[...], preferred_element_type=jnp.float32)
    es2, ed2 = _col_row(h2, a2s_ref[...], a2d_ref[...])
    att2 = _masked_softmax(_leaky(es2 + ed2), adj)
    mole = _elu(jnp.dot(att2, h2, preferred_element_type=jnp.float32)) + mole

    h3 = jnp.dot(mole, W3_ref[...], preferred_element_type=jnp.float32)
    es3, ed3 = _col_row(h3, a3s_ref[...], a3d_ref[...])
    att3 = _masked_softmax(_leaky(es3 + ed3), adj)
    out = jnp.dot(att3, h3, preferred_element_type=jnp.float32)

    res = jax.lax.dot_general(
        fcw_ref[...], out, (((0,), (0,)), ((), ())),
        preferred_element_type=jnp.float32)
    o_ref[0] = res + fcb_ref[0, 0]


def kernel(atom_feature, weight, adj, W1, a1s, a1d, W2, a2s, a2d, W3, a3s, a3d,
           fc_w, fc_b):
    del weight

    W1c = jnp.transpose(W1, (1, 0, 2)).reshape(DIN, HID)
    block_eye = jnp.repeat(jnp.eye(NH, dtype=jnp.float32), DH, axis=0)
    A1s = block_eye * a1s.reshape(HID)[:, None]
    A1d = block_eye * a1d.reshape(HID)[:, None]
    a2s_c = a2s[:, None]
    a2d_c = a2d[:, None]
    a3s_c = a3s[:, None]
    a3d_c = a3d[:, None]
    fcw_c = fc_w[:, None]
    fcb = fc_b.reshape(1, 1)

    const = lambda b: (0, 0)
    grid = (B,)
    out = pl.pallas_call(
        _gat_kernel,
        grid=grid,
        in_specs=[
            pl.BlockSpec((1, N, DIN), lambda b: (b, 0, 0)),
            pl.BlockSpec((1, N, N), lambda b: (b, 0, 0)),
            pl.BlockSpec((DIN, HID), const),
            pl.BlockSpec((HID, NH), const),
            pl.BlockSpec((HID, NH), const),
            pl.BlockSpec((HID, HID), const),
            pl.BlockSpec((HID, 1), const),
            pl.BlockSpec((HID, 1), const),
            pl.BlockSpec((HID, DOUT), const),
            pl.BlockSpec((DOUT, 1), const),
            pl.BlockSpec((DOUT, 1), const),
            pl.BlockSpec((N, 1), const),
            pl.BlockSpec((1, 1), const),
        ],
        out_specs=pl.BlockSpec((1, 1, DOUT), lambda b: (b, 0, 0)),
        out_shape=jax.ShapeDtypeStruct((B, 1, DOUT), jnp.float32),
        compiler_params=pltpu.CompilerParams(
            dimension_semantics=("parallel",)),
    )(atom_feature, adj, W1c, A1s, A1d, W2, a2s_c, a2d_c, W3, a3s_c, a3d_c,
      fcw_c, fcb)
    return out.reshape(B, DOUT)


# speedup vs baseline: 2.6697x; 2.6697x over previous
import jax
import jax.numpy as jnp
from jax.experimental import pallas as pl
from jax.experimental.pallas import tpu as pltpu

B, N, DIN, NH, DH = 256, 128, 128, 8, 16
HID = NH * DH
DOUT = 512

_NEG = -9e15
G = 4


def _leaky(x):
    return jnp.where(x > 0, x, 0.2 * x)


def _elu(x):
    return jnp.where(x > 0, x, jnp.exp(jnp.minimum(x, 0.0)) - 1.0)


def _softmax_rows(e):
    m = jnp.max(e, axis=-1, keepdims=True)
    p = jnp.exp(e - m)
    return p / jnp.sum(p, axis=-1, keepdims=True)


def _gat_kernel(x_ref, adj_ref, W1c_ref, A1s_ref, A1d_ref, HM_ref,
                W2_ref, a2s_ref, a2d_ref, W3_ref, a3s_ref, a3d_ref,
                FCW_ref, fcb_ref, o_ref):
    adjs = [adj_ref[g] for g in range(G)]

    x_all = x_ref[...].reshape(G * N, DIN)
    h1_all = jnp.dot(x_all, W1c_ref[...], preferred_element_type=jnp.float32)
    es_all = jnp.dot(h1_all, A1s_ref[...], preferred_element_type=jnp.float32)
    ed_all = jax.lax.dot_general(
        A1d_ref[...], h1_all, (((0,), (1,)), ((), ())),
        preferred_element_type=jnp.float32)

    blocks = []
    for g in range(G):
        for i in range(NH):
            e = _leaky(es_all[g * N:(g + 1) * N, i:i + 1]
                       + ed_all[i:i + 1, g * N:(g + 1) * N])
            blocks.append(jnp.where(adjs[g] > 0, e, _NEG))
    att1 = _softmax_rows(jnp.concatenate(blocks, axis=0))

    hm = HM_ref[...][:, None, :]
    moles = []
    for g in range(G):
        og = jnp.dot(att1[g * NH * N:(g + 1) * NH * N, :],
                     h1_all[g * N:(g + 1) * N, :],
                     preferred_element_type=jnp.float32)
        moles.append(jnp.sum(og.reshape(NH, N, HID) * hm, axis=0))
    mole_all = _elu(jnp.concatenate(moles, axis=0))

    h2_all = jnp.dot(mole_all, W2_ref[...], preferred_element_type=jnp.float32)
    es2 = jnp.dot(h2_all, a2s_ref[...], preferred_element_type=jnp.float32)
    ed2 = jax.lax.dot_general(
        a2d_ref[...], h2_all, (((0,), (1,)), ((), ())),
        preferred_element_type=jnp.float32)
    blocks = []
    for g in range(G):
        e = _leaky(es2[g * N:(g + 1) * N, :] + ed2[:, g * N:(g + 1) * N])
        blocks.append(jnp.where(adjs[g] > 0, e, _NEG))
    att2 = _softmax_rows(jnp.concatenate(blocks, axis=0))
    aggs = [jnp.dot(att2[g * N:(g + 1) * N, :], h2_all[g * N:(g + 1) * N, :],
                    preferred_element_type=jnp.float32)
            for g in range(G)]
    mole_all = _elu(jnp.concatenate(aggs, axis=0)) + mole_all

    h3_all = jnp.dot(mole_all, W3_ref[...], preferred_element_type=jnp.float32)
    es3 = jnp.dot(h3_all, a3s_ref[...], preferred_element_type=jnp.float32)
    ed3 = jax.lax.dot_general(
        a3d_ref[...], h3_all, (((0,), (1,)), ((), ())),
        preferred_element_type=jnp.float32)
    blocks = []
    for g in range(G):
        e = _leaky(es3[g * N:(g + 1) * N, :] + ed3[:, g * N:(g + 1) * N])
        blocks.append(jnp.where(adjs[g] > 0, e, _NEG))
    att3 = _softmax_rows(jnp.concatenate(blocks, axis=0))
    outs = [jnp.dot(att3[g * N:(g + 1) * N, :], h3_all[g * N:(g + 1) * N, :],
                    preferred_element_type=jnp.float32)
            for g in range(G)]
    out_all = jnp.concatenate(outs, axis=0)

    res = jnp.dot(FCW_ref[...], out_all, preferred_element_type=jnp.float32)
    res = res + fcb_ref[0, 0]
    for g in range(G):
        o_ref[g] = res[g:g + 1, :]


def kernel(atom_feature, weight, adj, W1, a1s, a1d, W2, a2s, a2d, W3, a3s, a3d,
           fc_w, fc_b):
    del weight

    W1c = jnp.transpose(W1, (1, 0, 2)).reshape(DIN, HID)
    block_eye = jnp.repeat(jnp.eye(NH, dtype=jnp.float32), DH, axis=0)
    A1s = block_eye * a1s.reshape(HID)[:, None]
    A1d = block_eye * a1d.reshape(HID)[:, None]
    HM = block_eye.T
    a2s_c = a2s[:, None]
    a2d_c = a2d[:, None]
    a3s_c = a3s[:, None]
    a3d_c = a3d[:, None]
    FCW = jnp.kron(jnp.eye(G, dtype=jnp.float32), fc_w[None, :])
    fcb = fc_b.reshape(1, 1)

    const = lambda b: (0, 0)
    grid = (B // G,)
    out = pl.pallas_call(
        _gat_kernel,
        grid=grid,
        in_specs=[
            pl.BlockSpec((G, N, DIN), lambda b: (b, 0, 0)),
            pl.BlockSpec((G, N, N), lambda b: (b, 0, 0)),
            pl.BlockSpec((DIN, HID), const),
            pl.BlockSpec((HID, NH), const),
            pl.BlockSpec((HID, NH), const),
            pl.BlockSpec((NH, HID), const),
            pl.BlockSpec((HID, HID), const),
            pl.BlockSpec((HID, 1), const),
            pl.BlockSpec((HID, 1), const),
            pl.BlockSpec((HID, DOUT), const),
            pl.BlockSpec((DOUT, 1), const),
            pl.BlockSpec((DOUT, 1), const),
            pl.BlockSpec((G, G * N), const),
            pl.BlockSpec((1, 1), const),
        ],
        out_specs=pl.BlockSpec((G, 1, DOUT), lambda b: (b, 0, 0)),
        out_shape=jax.ShapeDtypeStruct((B, 1, DOUT), jnp.float32),
        compiler_params=pltpu.CompilerParams(
            dimension_semantics=("parallel",)),
    )(atom_feature, adj, W1c, A1s, A1d, HM, W2, a2s_c, a2d_c, W3, a3s_c,
      a3d_c, FCW, fcb)
    return out.reshape(B, DOUT)


# speedup vs baseline: 3.4872x; 1.3062x over previous
import jax
import jax.numpy as jnp
from jax.experimental import pallas as pl
from jax.experimental.pallas import tpu as pltpu

B, N, DIN, NH, DH = 256, 128, 128, 8, 16
HID = NH * DH
DOUT = 512

_NEG = -9e15
G = 8


def _leaky(x):
    return jnp.where(x > 0, x, 0.2 * x)


def _elu(x):
    return jnp.where(x > 0, x, jnp.exp(jnp.minimum(x, 0.0)) - 1.0)


def _softmax_rows(e):
    m = jnp.max(e, axis=-1, keepdims=True)
    p = jnp.exp(e - m)
    return p / jnp.sum(p, axis=-1, keepdims=True)


def _gat_kernel(x_ref, adj_ref, W1c_ref, A1s_ref, A1d_ref, HM_ref,
                W2_ref, a2s_ref, a2d_ref, W3_ref, a3s_ref, a3d_ref,
                FCW_ref, fcb_ref, o_ref):
    adjs = [adj_ref[g] for g in range(G)]

    x_all = x_ref[...].reshape(G * N, DIN)
    h1_all = jnp.dot(x_all, W1c_ref[...], preferred_element_type=jnp.float32)
    es_all = jnp.dot(h1_all, A1s_ref[...], preferred_element_type=jnp.float32)
    ed_all = jax.lax.dot_general(
        A1d_ref[...], h1_all, (((0,), (1,)), ((), ())),
        preferred_element_type=jnp.float32)

    blocks = []
    for g in range(G):
        for i in range(NH):
            e = _leaky(es_all[g * N:(g + 1) * N, i:i + 1]
                       + ed_all[i:i + 1, g * N:(g + 1) * N])
            blocks.append(jnp.where(adjs[g] > 0, e, _NEG))
    att1 = _softmax_rows(jnp.concatenate(blocks, axis=0))

    hm = HM_ref[...][:, None, :]
    moles = []
    for g in range(G):
        og = jnp.dot(att1[g * NH * N:(g + 1) * NH * N, :],
                     h1_all[g * N:(g + 1) * N, :],
                     preferred_element_type=jnp.float32)
        moles.append(jnp.sum(og.reshape(NH, N, HID) * hm, axis=0))
    mole_all = _elu(jnp.concatenate(moles, axis=0))

    h2_all = jnp.dot(mole_all, W2_ref[...], preferred_element_type=jnp.float32)
    es2 = jnp.dot(h2_all, a2s_ref[...], preferred_element_type=jnp.float32)
    ed2 = jax.lax.dot_general(
        a2d_ref[...], h2_all, (((0,), (1,)), ((), ())),
        preferred_element_type=jnp.float32)
    blocks = []
    for g in range(G):
        e = _leaky(es2[g * N:(g + 1) * N, :] + ed2[:, g * N:(g + 1) * N])
        blocks.append(jnp.where(adjs[g] > 0, e, _NEG))
    att2 = _softmax_rows(jnp.concatenate(blocks, axis=0))
    aggs = [jnp.dot(att2[g * N:(g + 1) * N, :], h2_all[g * N:(g + 1) * N, :],
                    preferred_element_type=jnp.float32)
            for g in range(G)]
    mole_all = _elu(jnp.concatenate(aggs, axis=0)) + mole_all

    h3_all = jnp.dot(mole_all, W3_ref[...], preferred_element_type=jnp.float32)
    es3 = jnp.dot(h3_all, a3s_ref[...], preferred_element_type=jnp.float32)
    ed3 = jax.lax.dot_general(
        a3d_ref[...], h3_all, (((0,), (1,)), ((), ())),
        preferred_element_type=jnp.float32)
    blocks = []
    for g in range(G):
        e = _leaky(es3[g * N:(g + 1) * N, :] + ed3[:, g * N:(g + 1) * N])
        blocks.append(jnp.where(adjs[g] > 0, e, _NEG))
    att3 = _softmax_rows(jnp.concatenate(blocks, axis=0))
    outs = [jnp.dot(att3[g * N:(g + 1) * N, :], h3_all[g * N:(g + 1) * N, :],
                    preferred_element_type=jnp.float32)
            for g in range(G)]
    out_all = jnp.concatenate(outs, axis=0)

    res = jnp.dot(FCW_ref[...], out_all, preferred_element_type=jnp.float32)
    res = res + fcb_ref[0, 0]
    for g in range(G):
        o_ref[g] = res[g:g + 1, :]


def kernel(atom_feature, weight, adj, W1, a1s, a1d, W2, a2s, a2d, W3, a3s, a3d,
           fc_w, fc_b):
    del weight

    W1c = jnp.transpose(W1, (1, 0, 2)).reshape(DIN, HID)
    block_eye = jnp.repeat(jnp.eye(NH, dtype=jnp.float32), DH, axis=0)
    A1s = block_eye * a1s.reshape(HID)[:, None]
    A1d = block_eye * a1d.reshape(HID)[:, None]
    HM = block_eye.T
    a2s_c = a2s[:, None]
    a2d_c = a2d[:, None]
    a3s_c = a3s[:, None]
    a3d_c = a3d[:, None]
    FCW = jnp.kron(jnp.eye(G, dtype=jnp.float32), fc_w[None, :])
    fcb = fc_b.reshape(1, 1)

    const = lambda b: (0, 0)
    grid = (B // G,)
    out = pl.pallas_call(
        _gat_kernel,
        grid=grid,
        in_specs=[
            pl.BlockSpec((G, N, DIN), lambda b: (b, 0, 0)),
            pl.BlockSpec((G, N, N), lambda b: (b, 0, 0)),
            pl.BlockSpec((DIN, HID), const),
            pl.BlockSpec((HID, NH), const),
            pl.BlockSpec((HID, NH), const),
            pl.BlockSpec((NH, HID), const),
            pl.BlockSpec((HID, HID), const),
            pl.BlockSpec((HID, 1), const),
            pl.BlockSpec((HID, 1), const),
            pl.BlockSpec((HID, DOUT), const),
            pl.BlockSpec((DOUT, 1), const),
            pl.BlockSpec((DOUT, 1), const),
            pl.BlockSpec((G, G * N), const),
            pl.BlockSpec((1, 1), const),
        ],
        out_specs=pl.BlockSpec((G, 1, DOUT), lambda b: (b, 0, 0)),
        out_shape=jax.ShapeDtypeStruct((B, 1, DOUT), jnp.float32),
        compiler_params=pltpu.CompilerParams(
            dimension_semantics=("parallel",)),
    )(atom_feature, adj, W1c, A1s, A1d, HM, W2, a2s_c, a2d_c, W3, a3s_c,
      a3d_c, FCW, fcb)
    return out.reshape(B, DOUT)


# speedup vs baseline: 3.9297x; 1.1269x over previous
import jax
import jax.numpy as jnp
from jax.experimental import pallas as pl
from jax.experimental.pallas import tpu as pltpu

B, N, DIN, NH, DH = 256, 128, 128, 8, 16
HID = NH * DH
DOUT = 512

_NEG = -9e15
G = 16


def _leaky(x):
    return jnp.where(x > 0, x, 0.2 * x)


def _elu(x):
    return jnp.where(x > 0, x, jnp.exp(jnp.minimum(x, 0.0)) - 1.0)


def _softmax_rows(e):
    m = jnp.max(e, axis=-1, keepdims=True)
    p = jnp.exp(e - m)
    return p / jnp.sum(p, axis=-1, keepdims=True)


def _gat_kernel(x_ref, adj_ref, W1c_ref, A1s_ref, A1d_ref, HM_ref,
                W2_ref, a2s_ref, a2d_ref, W3_ref, a3s_ref, a3d_ref,
                FCW_ref, fcb_ref, o_ref):
    adjs = [adj_ref[g] for g in range(G)]

    x_all = x_ref[...].reshape(G * N, DIN)
    h1_all = jnp.dot(x_all, W1c_ref[...], preferred_element_type=jnp.float32)
    es_all = jnp.dot(h1_all, A1s_ref[...], preferred_element_type=jnp.float32)
    ed_all = jax.lax.dot_general(
        A1d_ref[...], h1_all, (((0,), (1,)), ((), ())),
        preferred_element_type=jnp.float32)

    blocks = []
    for g in range(G):
        for i in range(NH):
            e = _leaky(es_all[g * N:(g + 1) * N, i:i + 1]
                       + ed_all[i:i + 1, g * N:(g + 1) * N])
            blocks.append(jnp.where(adjs[g] > 0, e, _NEG))
    att1 = _softmax_rows(jnp.concatenate(blocks, axis=0))

    hm = HM_ref[...][:, None, :]
    moles = []
    for g in range(G):
        og = jnp.dot(att1[g * NH * N:(g + 1) * NH * N, :],
                     h1_all[g * N:(g + 1) * N, :],
                     preferred_element_type=jnp.float32)
        moles.append(jnp.sum(og.reshape(NH, N, HID) * hm, axis=0))
    mole_all = _elu(jnp.concatenate(moles, axis=0))

    h2_all = jnp.dot(mole_all, W2_ref[...], preferred_element_type=jnp.float32)
    es2 = jnp.dot(h2_all, a2s_ref[...], preferred_element_type=jnp.float32)
    ed2 = jax.lax.dot_general(
        a2d_ref[...], h2_all, (((0,), (1,)), ((), ())),
        preferred_element_type=jnp.float32)
    blocks = []
    for g in range(G):
        e = _leaky(es2[g * N:(g + 1) * N, :] + ed2[:, g * N:(g + 1) * N])
        blocks.append(jnp.where(adjs[g] > 0, e, _NEG))
    att2 = _softmax_rows(jnp.concatenate(blocks, axis=0))
    aggs = [jnp.dot(att2[g * N:(g + 1) * N, :], h2_all[g * N:(g + 1) * N, :],
                    preferred_element_type=jnp.float32)
            for g in range(G)]
    mole_all = _elu(jnp.concatenate(aggs, axis=0)) + mole_all

    h3_all = jnp.dot(mole_all, W3_ref[...], preferred_element_type=jnp.float32)
    es3 = jnp.dot(h3_all, a3s_ref[...], preferred_element_type=jnp.float32)
    ed3 = jax.lax.dot_general(
        a3d_ref[...], h3_all, (((0,), (1,)), ((), ())),
        preferred_element_type=jnp.float32)
    blocks = []
    for g in range(G):
        e = _leaky(es3[g * N:(g + 1) * N, :] + ed3[:, g * N:(g + 1) * N])
        blocks.append(jnp.where(adjs[g] > 0, e, _NEG))
    att3 = _softmax_rows(jnp.concatenate(blocks, axis=0))
    outs = [jnp.dot(att3[g * N:(g + 1) * N, :], h3_all[g * N:(g + 1) * N, :],
                    preferred_element_type=jnp.float32)
            for g in range(G)]
    out_all = jnp.concatenate(outs, axis=0)

    res = jnp.dot(FCW_ref[...], out_all, preferred_element_type=jnp.float32)
    res = res + fcb_ref[0, 0]
    for g in range(G):
        o_ref[g] = res[g:g + 1, :]


def kernel(atom_feature, weight, adj, W1, a1s, a1d, W2, a2s, a2d, W3, a3s, a3d,
           fc_w, fc_b):
    del weight

    W1c = jnp.transpose(W1, (1, 0, 2)).reshape(DIN, HID)
    block_eye = jnp.repeat(jnp.eye(NH, dtype=jnp.float32), DH, axis=0)
    A1s = block_eye * a1s.reshape(HID)[:, None]
    A1d = block_eye * a1d.reshape(HID)[:, None]
    HM = block_eye.T
    a2s_c = a2s[:, None]
    a2d_c = a2d[:, None]
    a3s_c = a3s[:, None]
    a3d_c = a3d[:, None]
    FCW = jnp.kron(jnp.eye(G, dtype=jnp.float32), fc_w[None, :])
    fcb = fc_b.reshape(1, 1)

    const = lambda b: (0, 0)
    grid = (B // G,)
    out = pl.pallas_call(
        _gat_kernel,
        grid=grid,
        in_specs=[
            pl.BlockSpec((G, N, DIN), lambda b: (b, 0, 0)),
            pl.BlockSpec((G, N, N), lambda b: (b, 0, 0)),
            pl.BlockSpec((DIN, HID), const),
            pl.BlockSpec((HID, NH), const),
            pl.BlockSpec((HID, NH), const),
            pl.BlockSpec((NH, HID), const),
            pl.BlockSpec((HID, HID), const),
            pl.BlockSpec((HID, 1), const),
            pl.BlockSpec((HID, 1), const),
            pl.BlockSpec((HID, DOUT), const),
            pl.BlockSpec((DOUT, 1), const),
            pl.BlockSpec((DOUT, 1), const),
            pl.BlockSpec((G, G * N), const),
            pl.BlockSpec((1, 1), const),
        ],
        out_specs=pl.BlockSpec((G, 1, DOUT), lambda b: (b, 0, 0)),
        out_shape=jax.ShapeDtypeStruct((B, 1, DOUT), jnp.float32),
        compiler_params=pltpu.CompilerParams(
            dimension_semantics=("parallel",)),
    )(atom_feature, adj, W1c, A1s, A1d, HM, W2, a2s_c, a2d_c, W3, a3s_c,
      a3d_c, FCW, fcb)
    return out.reshape(B, DOUT)


# speedup vs baseline: 4.4441x; 1.1309x over previous
import jax
import jax.numpy as jnp
from jax.experimental import pallas as pl
from jax.experimental.pallas import tpu as pltpu

B, N, DIN, NH, DH = 256, 128, 128, 8, 16
HID = NH * DH
DOUT = 512

G = 16


def _leaky(x):
    return jnp.where(x > 0, x, 0.2 * x)


def _elu(x):
    return jnp.where(x > 0, x, jnp.exp(jnp.minimum(x, 0.0)) - 1.0)


def _norm_rows(p):
    return p / jnp.sum(p, axis=-1, keepdims=True)


def _gat_kernel(x_ref, adj_ref, W1c_ref, A1s_ref, A1d_ref, HM_ref,
                W2_ref, a2s_ref, a2d_ref, W3_ref, a3s_ref, a3d_ref,
                FCW_ref, fcb_ref, o_ref):
    adjs = [adj_ref[g] for g in range(G)]

    x_all = x_ref[...].reshape(G * N, DIN)
    h1_all = jnp.dot(x_all, W1c_ref[...], preferred_element_type=jnp.float32)
    es_all = jnp.dot(h1_all, A1s_ref[...], preferred_element_type=jnp.float32)
    ed_all = jax.lax.dot_general(
        A1d_ref[...], h1_all, (((0,), (1,)), ((), ())),
        preferred_element_type=jnp.float32)

    blocks = []
    for g in range(G):
        for i in range(NH):
            e = _leaky(es_all[g * N:(g + 1) * N, i:i + 1]
                       + ed_all[i:i + 1, g * N:(g + 1) * N])
            blocks.append(jnp.exp(e) * adjs[g])
    att1 = _norm_rows(jnp.concatenate(blocks, axis=0))

    hm = HM_ref[...][:, None, :]
    moles = []
    for g in range(G):
        og = jnp.dot(att1[g * NH * N:(g + 1) * NH * N, :],
                     h1_all[g * N:(g + 1) * N, :],
                     preferred_element_type=jnp.float32)
        moles.append(jnp.sum(og.reshape(NH, N, HID) * hm, axis=0))
    mole_all = _elu(jnp.concatenate(moles, axis=0))

    h2_all = jnp.dot(mole_all, W2_ref[...], preferred_element_type=jnp.float32)
    es2 = jnp.dot(h2_all, a2s_ref[...], preferred_element_type=jnp.float32)
    ed2 = jax.lax.dot_general(
        a2d_ref[...], h2_all, (((0,), (1,)), ((), ())),
        preferred_element_type=jnp.float32)
    blocks = []
    for g in range(G):
        e = _leaky(es2[g * N:(g + 1) * N, :] + ed2[:, g * N:(g + 1) * N])
        blocks.append(jnp.exp(e) * adjs[g])
    att2 = _norm_rows(jnp.concatenate(blocks, axis=0))
    aggs = [jnp.dot(att2[g * N:(g + 1) * N, :], h2_all[g * N:(g + 1) * N, :],
                    preferred_element_type=jnp.float32)
            for g in range(G)]
    mole_all = _elu(jnp.concatenate(aggs, axis=0)) + mole_all

    h3_all = jnp.dot(mole_all, W3_ref[...], preferred_element_type=jnp.float32)
    es3 = jnp.dot(h3_all, a3s_ref[...], preferred_element_type=jnp.float32)
    ed3 = jax.lax.dot_general(
        a3d_ref[...], h3_all, (((0,), (1,)), ((), ())),
        preferred_element_type=jnp.float32)
    blocks = []
    for g in range(G):
        e = _leaky(es3[g * N:(g + 1) * N, :] + ed3[:, g * N:(g + 1) * N])
        blocks.append(jnp.exp(e) * adjs[g])
    att3 = _norm_rows(jnp.concatenate(blocks, axis=0))
    outs = [jnp.dot(att3[g * N:(g + 1) * N, :], h3_all[g * N:(g + 1) * N, :],
                    preferred_element_type=jnp.float32)
            for g in range(G)]
    out_all = jnp.concatenate(outs, axis=0)

    res = jnp.dot(FCW_ref[...], out_all, preferred_element_type=jnp.float32)
    res = res + fcb_ref[0, 0]
    for g in range(G):
        o_ref[g] = res[g:g + 1, :]


def kernel(atom_feature, weight, adj, W1, a1s, a1d, W2, a2s, a2d, W3, a3s, a3d,
           fc_w, fc_b):
    del weight

    W1c = jnp.transpose(W1, (1, 0, 2)).reshape(DIN, HID)
    block_eye = jnp.repeat(jnp.eye(NH, dtype=jnp.float32), DH, axis=0)
    A1s = block_eye * a1s.reshape(HID)[:, None]
    A1d = block_eye * a1d.reshape(HID)[:, None]
    HM = block_eye.T
    a2s_c = a2s[:, None]
    a2d_c = a2d[:, None]
    a3s_c = a3s[:, None]
    a3d_c = a3d[:, None]
    FCW = jnp.kron(jnp.eye(G, dtype=jnp.float32), fc_w[None, :])
    fcb = fc_b.reshape(1, 1)

    const = lambda b: (0, 0)
    grid = (B // G,)
    out = pl.pallas_call(
        _gat_kernel,
        grid=grid,
        in_specs=[
            pl.BlockSpec((G, N, DIN), lambda b: (b, 0, 0)),
            pl.BlockSpec((G, N, N), lambda b: (b, 0, 0)),
            pl.BlockSpec((DIN, HID), const),
            pl.BlockSpec((HID, NH), const),
            pl.BlockSpec((HID, NH), const),
            pl.BlockSpec((NH, HID), const),
            pl.BlockSpec((HID, HID), const),
            pl.BlockSpec((HID, 1), const),
            pl.BlockSpec((HID, 1), const),
            pl.BlockSpec((HID, DOUT), const),
            pl.BlockSpec((DOUT, 1), const),
            pl.BlockSpec((DOUT, 1), const),
            pl.BlockSpec((G, G * N), const),
            pl.BlockSpec((1, 1), const),
        ],
        out_specs=pl.BlockSpec((G, 1, DOUT), lambda b: (b, 0, 0)),
        out_shape=jax.ShapeDtypeStruct((B, 1, DOUT), jnp.float32),
        compiler_params=pltpu.CompilerParams(
            dimension_semantics=("parallel",)),
    )(atom_feature, adj, W1c, A1s, A1d, HM, W2, a2s_c, a2d_c, W3, a3s_c,
      a3d_c, FCW, fcb)
    return out.reshape(B, DOUT)


# speedup vs baseline: 4.7143x; 1.0608x over previous
import jax
import jax.numpy as jnp
from jax.experimental import pallas as pl
from jax.experimental.pallas import tpu as pltpu

B, N, DIN, NH, DH = 256, 128, 128, 8, 16
HID = NH * DH
DOUT = 512

G = 16


def _leaky(x):
    return jnp.maximum(x, 0.2 * x)


def _bf(x):
    return x.astype(jnp.bfloat16)


def _elu(x):
    return jnp.where(x > 0, x, jnp.exp(jnp.minimum(x, 0.0)) - 1.0)


def _norm_rows(p):
    return p / jnp.sum(p, axis=-1, keepdims=True)


def _gat_kernel(x_ref, adj_ref, W1c_ref, A1s_ref, A1d_ref, HM_ref,
                W2_ref, a2s_ref, a2d_ref, W3_ref, a3s_ref, a3d_ref,
                FCW_ref, fcb_ref, o_ref):
    adjs = [adj_ref[g] for g in range(G)]

    x_all = x_ref[...].reshape(G * N, DIN)
    h1_all = jnp.dot(_bf(x_all), _bf(W1c_ref[...]),
                     preferred_element_type=jnp.float32)
    es_all = jnp.dot(h1_all, A1s_ref[...], preferred_element_type=jnp.float32)
    ed_all = jax.lax.dot_general(
        A1d_ref[...], h1_all, (((0,), (1,)), ((), ())),
        preferred_element_type=jnp.float32)

    blocks = []
    for g in range(G):
        for i in range(NH):
            e = _leaky(es_all[g * N:(g + 1) * N, i:i + 1]
                       + ed_all[i:i + 1, g * N:(g + 1) * N])
            blocks.append(jnp.exp(e) * adjs[g])
    att1 = _norm_rows(jnp.concatenate(blocks, axis=0))

    hm = HM_ref[...][:, None, :]
    att1 = _bf(att1)
    h1_b = _bf(h1_all)
    moles = []
    for g in range(G):
        og = jnp.dot(att1[g * NH * N:(g + 1) * NH * N, :],
                     h1_b[g * N:(g + 1) * N, :],
                     preferred_element_type=jnp.float32)
        moles.append(jnp.sum(og.reshape(NH, N, HID) * hm, axis=0))
    mole_all = _elu(jnp.concatenate(moles, axis=0))

    h2_all = jnp.dot(_bf(mole_all), _bf(W2_ref[...]),
                     preferred_element_type=jnp.float32)
    es2 = jnp.dot(h2_all, a2s_ref[...], preferred_element_type=jnp.float32)
    ed2 = jax.lax.dot_general(
        a2d_ref[...], h2_all, (((0,), (1,)), ((), ())),
        preferred_element_type=jnp.float32)
    blocks = []
    for g in range(G):
        e = _leaky(es2[g * N:(g + 1) * N, :] + ed2[:, g * N:(g + 1) * N])
        blocks.append(jnp.exp(e) * adjs[g])
    att2 = _bf(_norm_rows(jnp.concatenate(blocks, axis=0)))
    h2_b = _bf(h2_all)
    aggs = [jnp.dot(att2[g * N:(g + 1) * N, :], h2_b[g * N:(g + 1) * N, :],
                    preferred_element_type=jnp.float32)
            for g in range(G)]
    mole_all = _elu(jnp.concatenate(aggs, axis=0)) + mole_all

    h3_all = jnp.dot(_bf(mole_all), _bf(W3_ref[...]),
                     preferred_element_type=jnp.float32)
    es3 = jnp.dot(h3_all, a3s_ref[...], preferred_element_type=jnp.float32)
    ed3 = jax.lax.dot_general(
        a3d_ref[...], h3_all, (((0,), (1,)), ((), ())),
        preferred_element_type=jnp.float32)
    blocks = []
    for g in range(G):
        e = _leaky(es3[g * N:(g + 1) * N, :] + ed3[:, g * N:(g + 1) * N])
        blocks.append(jnp.exp(e) * adjs[g])
    att3 = _bf(_norm_rows(jnp.concatenate(blocks, axis=0)))
    h3_b = _bf(h3_all)
    outs = [jnp.dot(att3[g * N:(g + 1) * N, :], h3_b[g * N:(g + 1) * N, :],
                    preferred_element_type=jnp.float32)
            for g in range(G)]
    out_all = jnp.concatenate(outs, axis=0)

    res = jnp.dot(FCW_ref[...], out_all, preferred_element_type=jnp.float32)
    res = res + fcb_ref[0, 0]
    for g in range(G):
        o_ref[g] = res[g:g + 1, :]


def kernel(atom_feature, weight, adj, W1, a1s, a1d, W2, a2s, a2d, W3, a3s, a3d,
           fc_w, fc_b):
    del weight

    W1c = jnp.transpose(W1, (1, 0, 2)).reshape(DIN, HID)
    block_eye = jnp.repeat(jnp.eye(NH, dtype=jnp.float32), DH, axis=0)
    A1s = block_eye * a1s.reshape(HID)[:, None]
    A1d = block_eye * a1d.reshape(HID)[:, None]
    HM = block_eye.T
    a2s_c = a2s[:, None]
    a2d_c = a2d[:, None]
    a3s_c = a3s[:, None]
    a3d_c = a3d[:, None]
    FCW = jnp.kron(jnp.eye(G, dtype=jnp.float32), fc_w[None, :])
    fcb = fc_b.reshape(1, 1)

    const = lambda b: (0, 0)
    grid = (B // G,)
    out = pl.pallas_call(
        _gat_kernel,
        grid=grid,
        in_specs=[
            pl.BlockSpec((G, N, DIN), lambda b: (b, 0, 0)),
            pl.BlockSpec((G, N, N), lambda b: (b, 0, 0)),
            pl.BlockSpec((DIN, HID), const),
            pl.BlockSpec((HID, NH), const),
            pl.BlockSpec((HID, NH), const),
            pl.BlockSpec((NH, HID), const),
            pl.BlockSpec((HID, HID), const),
            pl.BlockSpec((HID, 1), const),
            pl.BlockSpec((HID, 1), const),
            pl.BlockSpec((HID, DOUT), const),
            pl.BlockSpec((DOUT, 1), const),
            pl.BlockSpec((DOUT, 1), const),
            pl.BlockSpec((G, G * N), const),
            pl.BlockSpec((1, 1), const),
        ],
        out_specs=pl.BlockSpec((G, 1, DOUT), lambda b: (b, 0, 0)),
        out_shape=jax.ShapeDtypeStruct((B, 1, DOUT), jnp.float32),
        compiler_params=pltpu.CompilerParams(
            dimension_semantics=("parallel",)),
    )(atom_feature, adj, W1c, A1s, A1d, HM, W2, a2s_c, a2d_c, W3, a3s_c,
      a3d_c, FCW, fcb)
    return out.reshape(B, DOUT)
